# baseline (speedup 1.0000x reference)
import functools

import jax
import jax.numpy as jnp
from jax import lax
from jax.experimental import pallas as pl
from jax.experimental.pallas import tpu as pltpu
from jax.experimental.pallas import tpu_sc as plsc

_B, _N, _DIMS, _K, _F = 2, 4096, 2, 16, 128
_QT = 256
_NC, _NS = 2, 16
_NW = _NC * _NS
_M = _B * _N * _K
_CH = 128
_CPW = _M // (_NW * _CH)


_G = 16
_GS = _N // _G
_PL = _G * _K


def _knn_body(pcq_ref, pcc_ref, idx_ref):
    b_id = pl.program_id(0)
    qx = pcq_ref[0, 0, :]
    qy = pcq_ref[0, 1, :]
    cxg = pcc_ref[0, 0]
    cyg = pcc_ref[0, 1]
    dx = cxg[:, :, None] - qx[None, None, :]
    dy = cyg[:, :, None] - qy[None, None, :]
    dist0 = dx * dx + dy * dy

    liota = lax.broadcasted_iota(jnp.int32, (_G, _GS, _QT), 1)
    goff = lax.broadcasted_iota(jnp.int32, (_G, _QT), 0) * _GS
    priota = lax.broadcasted_iota(jnp.int32, (_PL, _QT), 0)
    big_i = jnp.int32(2 ** 30)
    inf = jnp.float32(jnp.inf)

    def cond(carry):
        s, _, _, _, allsafe = carry
        return jnp.logical_and(s < _K, jnp.logical_not(allsafe))

    def sweep(carry):
        s, dist, poolv, pooli, _ = carry
        m = jnp.min(dist, axis=1, keepdims=True)
        sel = jnp.where(dist == m, liota, big_i)
        lpos = jnp.min(sel, axis=1, keepdims=True)
        onehot = sel == lpos
        dist = jnp.where(onehot, inf, dist)
        mv = m.reshape(_G, _QT)
        gi = lpos.reshape(_G, _QT) + goff
        inwin = jnp.logical_and(priota >= s * _G, priota < (s + 1) * _G)
        poolv = jnp.where(inwin, jnp.tile(mv, (_K, 1)), poolv)
        pooli = jnp.where(inwin, jnp.tile(gi, (_K, 1)), pooli)
        cnt = jnp.sum((poolv[:, None, :] < mv[None, :, :]).astype(jnp.int32),
                      axis=0)
        allsafe = jnp.all(cnt >= _K)
        return s + 1, dist, poolv, pooli, allsafe

    poolv0 = jnp.full((_PL, _QT), inf, jnp.float32)
    pooli0 = jnp.full((_PL, _QT), big_i, jnp.int32)
    _, _, poolv, pooli, _ = lax.while_loop(
        cond, sweep,
        (jnp.int32(0), dist0, poolv0, pooli0, jnp.bool_(False)))

    ids = []
    for _ in range(_K):
        mk = jnp.min(poolv, axis=0, keepdims=True)
        selk = jnp.where(poolv == mk, pooli, big_i)
        gidx = jnp.min(selk, axis=0, keepdims=True)
        oh = selk == gidx
        poolv = jnp.where(oh, inf, poolv)
        ids.append(gidx[0])

    nid = jnp.stack(ids, axis=0)
    idx_ref[0] = nid + b_id * _N


def _knn(pcT):
    return pl.pallas_call(
        _knn_body,
        grid=(_B, _N // _QT),
        in_specs=[
            pl.BlockSpec((1, _DIMS, _QT), lambda b, i: (b, 0, i)),
            pl.BlockSpec((1, _DIMS, _G, _GS), lambda b, i: (b, 0, 0, 0)),
        ],
        out_specs=pl.BlockSpec((1, _K, _QT), lambda b, i: (b, 0, i)),
        out_shape=jax.ShapeDtypeStruct((_B, _K, _N), jnp.int32),
    )(pcT, pcT.reshape(_B, _DIMS, _G, _GS))


@functools.cache
def _sc_gather():
    @functools.partial(
        pl.kernel,
        out_type=[
            jax.ShapeDtypeStruct((_M, 2 * _F), jnp.float32),
            jax.ShapeDtypeStruct((_M,), jnp.float32),
            jax.ShapeDtypeStruct((_M,), jnp.float32),
        ],
        mesh=plsc.VectorSubcoreMesh(
            core_axis_name="c", subcore_axis_name="s",
            num_cores=_NC, num_subcores=_NS,
        ),
        scratch_types=[
            pltpu.VMEM((_CH,), jnp.int32),
            pltpu.VMEM((_CH, _F), jnp.float32),
            pltpu.VMEM((2 * _B * _N,), jnp.float32),
            pltpu.VMEM((_CH,), jnp.float32),
            pltpu.VMEM((_CH,), jnp.float32),
            pltpu.SemaphoreType.DMA,
        ],
        compiler_params=pltpu.CompilerParams(needs_layout_passes=False),
    )
    def gather(tab_hbm, pc_hbm, idx_hbm, nf_hbm, npx_hbm, npy_hbm,
               idx_v, rows_v, pct_v, nx_v, ny_v, sem):
        wid = lax.axis_index("s") * _NC + lax.axis_index("c")
        pltpu.sync_copy(pc_hbm, pct_v)

        def chunk(c, _):
            row = wid * _CPW + c
            pltpu.sync_copy(idx_hbm.at[row], idx_v)
            cp1 = pltpu.async_copy(tab_hbm.at[idx_v], rows_v, sem)
            for j in range(_CH // 16):
                i16 = idx_v[pl.ds(j * 16, 16)] * 2
                nx_v[pl.ds(j * 16, 16)] = plsc.load_gather(pct_v, [i16])
                ny_v[pl.ds(j * 16, 16)] = plsc.load_gather(pct_v, [i16 + 1])
            cp1.wait()
            pltpu.sync_copy(rows_v,
                            nf_hbm.at[pl.ds(row * _CH, _CH), pl.ds(0, _F)])
            pltpu.sync_copy(nx_v, npx_hbm.at[pl.ds(row * _CH, _CH)])
            pltpu.sync_copy(ny_v, npy_hbm.at[pl.ds(row * _CH, _CH)])
            return ()

        lax.fori_loop(0, _CPW, chunk, (), unroll=False)

    return gather


def _mlp_body(pcq_ref, npx_ref, npy_ref, w_ref, b_ref, _nf_ref, r_ref):
    qx = pcq_ref[0, 0, :]
    qy = pcq_ref[0, 1, :]
    nx = npx_ref[...]
    ny = npy_ref[...]
    qxb = jnp.broadcast_to(qx[:, None], (_QT, _K))
    qyb = jnp.broadcast_to(qy[:, None], (_QT, _K))
    relx = qxb - nx
    rely = qyb - ny
    norm = jnp.sqrt(relx * relx + rely * rely + jnp.float32(1e-12))

    feats7 = (qxb, qyb, nx, ny, relx, rely, norm)
    acc = jnp.broadcast_to(b_ref[0][None, None, :], (_QT, _K, _F))
    for c in range(7):
        acc = acc + feats7[c][:, :, None] * w_ref[c][None, None, :]
    r_ref[0] = jnp.maximum(acc, jnp.float32(0.0))


def _mlp(pcT, npx, npy, W, b2, nf_full):
    qspec = pl.BlockSpec((_QT, _K), lambda b, i: (b * (_N // _QT) + i, 0))
    return pl.pallas_call(
        _mlp_body,
        grid=(_B, _N // _QT),
        in_specs=[
            pl.BlockSpec((1, _DIMS, _QT), lambda b, i: (b, 0, i)),
            qspec,
            qspec,
            pl.BlockSpec((7, _F), lambda b, i: (0, 0)),
            pl.BlockSpec((1, _F), lambda b, i: (0, 0)),
            pl.BlockSpec(memory_space=pl.ANY),
        ],
        out_specs=pl.BlockSpec((1, _QT, _K, _F), lambda b, i: (b, i, 0, 1)),
        out_shape=jax.ShapeDtypeStruct((_B, _N, _K, 2 * _F), jnp.float32),
        input_output_aliases={5: 0},
    )(pcT, npx, npy, W, b2, nf_full)


def kernel(pc, feats, W, b):
    pcT = pc.transpose(0, 2, 1)
    nid = _knn(pcT).transpose(0, 2, 1)
    tab = feats.reshape(_B * _N, _F)
    pc_flat = pc.reshape(2 * _B * _N)
    idx2 = nid.reshape(_M // _CH, _CH)
    nf_full, npx, npy = _sc_gather()(tab, pc_flat, idx2)
    return _mlp(pcT, npx.reshape(_B * _N, _K), npy.reshape(_B * _N, _K),
                W, b.reshape(1, _F),
                nf_full.reshape(_B, _N, _K, 2 * _F))


# speedup vs baseline: 1.0168x; 1.0168x over previous
import functools

import jax
import jax.numpy as jnp
from jax import lax
from jax.experimental import pallas as pl
from jax.experimental.pallas import tpu as pltpu
from jax.experimental.pallas import tpu_sc as plsc

_B, _N, _DIMS, _K, _F = 2, 4096, 2, 16, 128
_QT = 1024
_NC, _NS = 2, 16
_NW = _NC * _NS
_M = _B * _N * _K
_CH = 128
_CPW = _M // (_NW * _CH)


def _knn_body(pcq_ref, pcc_ref, idx_ref):
    b_id = pl.program_id(0)
    qx = pcq_ref[0, 0, :]
    qy = pcq_ref[0, 1, :]
    cx = pcc_ref[0, 0, :]
    cy = pcc_ref[0, 1, :]
    dx = qx[:, None] - cx[None, :]
    dy = qy[:, None] - cy[None, :]
    dist = dx * dx + dy * dy

    iota = lax.broadcasted_iota(jnp.int32, (_QT, _N), 1)
    big_i = jnp.int32(2 ** 30)
    inf = jnp.float32(jnp.inf)

    id_l = []
    for _ in range(_K):
        m = jnp.min(dist, axis=1, keepdims=True)
        sel = jnp.where(dist == m, iota, big_i)
        idx = jnp.min(sel, axis=1, keepdims=True)
        onehot = sel == idx
        id_l.append(idx[:, 0])
        dist = jnp.where(onehot, inf, dist)

    nid = jnp.stack(id_l, axis=1)
    idx_ref[0] = nid + b_id * _N


def _knn(pcT):
    return pl.pallas_call(
        _knn_body,
        grid=(_B, _N // _QT),
        compiler_params=pltpu.CompilerParams(
            vmem_limit_bytes=110 * 1024 * 1024),
        in_specs=[
            pl.BlockSpec((1, _DIMS, _QT), lambda b, i: (b, 0, i)),
            pl.BlockSpec((1, _DIMS, _N), lambda b, i: (b, 0, 0)),
        ],
        out_specs=pl.BlockSpec((1, _QT, _K), lambda b, i: (b, i, 0)),
        out_shape=jax.ShapeDtypeStruct((_B, _N, _K), jnp.int32),
    )(pcT, pcT)


@functools.cache
def _sc_gather():
    @functools.partial(
        pl.kernel,
        out_type=[
            jax.ShapeDtypeStruct((_M, 2 * _F), jnp.float32),
            jax.ShapeDtypeStruct((_M,), jnp.float32),
            jax.ShapeDtypeStruct((_M,), jnp.float32),
        ],
        mesh=plsc.VectorSubcoreMesh(
            core_axis_name="c", subcore_axis_name="s",
            num_cores=_NC, num_subcores=_NS,
        ),
        scratch_types=[
            pltpu.VMEM((_CH,), jnp.int32),
            pltpu.VMEM((_CH, _F), jnp.float32),
            pltpu.VMEM((2 * _B * _N,), jnp.float32),
            pltpu.VMEM((_CH,), jnp.float32),
            pltpu.VMEM((_CH,), jnp.float32),
            pltpu.SemaphoreType.DMA,
        ],
        compiler_params=pltpu.CompilerParams(needs_layout_passes=False),
    )
    def gather(tab_hbm, pc_hbm, idx_hbm, nf_hbm, npx_hbm, npy_hbm,
               idx_v, rows_v, pct_v, nx_v, ny_v, sem):
        wid = lax.axis_index("s") * _NC + lax.axis_index("c")
        pltpu.sync_copy(pc_hbm, pct_v)

        def chunk(c, _):
            row = wid * _CPW + c
            pltpu.sync_copy(idx_hbm.at[row], idx_v)
            cp1 = pltpu.async_copy(tab_hbm.at[idx_v], rows_v, sem)
            for j in range(_CH // 16):
                i16 = idx_v[pl.ds(j * 16, 16)] * 2
                nx_v[pl.ds(j * 16, 16)] = plsc.load_gather(pct_v, [i16])
                ny_v[pl.ds(j * 16, 16)] = plsc.load_gather(pct_v, [i16 + 1])
            cp1.wait()
            pltpu.sync_copy(rows_v,
                            nf_hbm.at[pl.ds(row * _CH, _CH), pl.ds(0, _F)])
            pltpu.sync_copy(nx_v, npx_hbm.at[pl.ds(row * _CH, _CH)])
            pltpu.sync_copy(ny_v, npy_hbm.at[pl.ds(row * _CH, _CH)])
            return ()

        lax.fori_loop(0, _CPW, chunk, (), unroll=False)

    return gather


def _mlp_body(pcq_ref, npx_ref, npy_ref, w_ref, b_ref, _nf_ref, r_ref):
    qx = pcq_ref[0, 0, :]
    qy = pcq_ref[0, 1, :]
    nx = npx_ref[...]
    ny = npy_ref[...]
    qxb = jnp.broadcast_to(qx[:, None], (_QT, _K))
    qyb = jnp.broadcast_to(qy[:, None], (_QT, _K))
    relx = qxb - nx
    rely = qyb - ny
    norm = jnp.sqrt(relx * relx + rely * rely + jnp.float32(1e-12))

    feats7 = (qxb, qyb, nx, ny, relx, rely, norm)
    acc = jnp.broadcast_to(b_ref[0][None, None, :], (_QT, _K, _F))
    for c in range(7):
        acc = acc + feats7[c][:, :, None] * w_ref[c][None, None, :]
    r_ref[0] = jnp.maximum(acc, jnp.float32(0.0))


def _mlp(pcT, npx, npy, W, b2, nf_full):
    qspec = pl.BlockSpec((_QT, _K), lambda b, i: (b * (_N // _QT) + i, 0))
    return pl.pallas_call(
        _mlp_body,
        grid=(_B, _N // _QT),
        in_specs=[
            pl.BlockSpec((1, _DIMS, _QT), lambda b, i: (b, 0, i)),
            qspec,
            qspec,
            pl.BlockSpec((7, _F), lambda b, i: (0, 0)),
            pl.BlockSpec((1, _F), lambda b, i: (0, 0)),
            pl.BlockSpec(memory_space=pl.ANY),
        ],
        out_specs=pl.BlockSpec((1, _QT, _K, _F), lambda b, i: (b, i, 0, 1)),
        out_shape=jax.ShapeDtypeStruct((_B, _N, _K, 2 * _F), jnp.float32),
        input_output_aliases={5: 0},
    )(pcT, npx, npy, W, b2, nf_full)


def kernel(pc, feats, W, b):
    pcT = pc.transpose(0, 2, 1)
    nid = _knn(pcT)
    tab = feats.reshape(_B * _N, _F)
    pc_flat = pc.reshape(2 * _B * _N)
    idx2 = nid.reshape(_M // _CH, _CH)
    nf_full, npx, npy = _sc_gather()(tab, pc_flat, idx2)
    return _mlp(pcT, npx.reshape(_B * _N, _K), npy.reshape(_B * _N, _K),
                W, b.reshape(1, _F),
                nf_full.reshape(_B, _N, _K, 2 * _F))


# speedup vs baseline: 1.1891x; 1.1694x over previous
import functools

import jax
import jax.numpy as jnp
from jax import lax
from jax.experimental import pallas as pl
from jax.experimental.pallas import tpu as pltpu
from jax.experimental.pallas import tpu_sc as plsc

_B, _N, _DIMS, _K, _F = 2, 4096, 2, 16, 128
_QT = 512
_NC, _NS = 2, 16
_NW = _NC * _NS
_M = _B * _N * _K
_CH = 128
_CPW = _M // (_NW * _CH)


def _knn_body(pcq_ref, pcc_ref, idx_ref):
    b_id = pl.program_id(0)
    qx = pcq_ref[0, 0, :]
    qy = pcq_ref[0, 1, :]
    cx = pcc_ref[0, 0, :]
    cy = pcc_ref[0, 1, :]
    dx = qx[:, None] - cx[None, :]
    dy = qy[:, None] - cy[None, :]
    dist = dx * dx + dy * dy

    iota = lax.broadcasted_iota(jnp.int32, (_QT, _N), 1)
    big_i = jnp.int32(2 ** 30)
    inf = jnp.float32(jnp.inf)

    t_id = pl.program_id(1)
    qbase = t_id * _QT
    rowi = lax.broadcasted_iota(jnp.int32, (_QT, _N), 0) + qbase
    dist = jnp.where(iota == rowi, inf, dist)
    id_l = [lax.iota(jnp.int32, _QT) + qbase]
    for _ in range(_K - 1):
        m = jnp.min(dist, axis=1, keepdims=True)
        sel = jnp.where(dist == m, iota, big_i)
        idx = jnp.min(sel, axis=1, keepdims=True)
        onehot = sel == idx
        id_l.append(idx[:, 0])
        dist = jnp.where(onehot, inf, dist)

    nid = jnp.stack(id_l, axis=1)
    idx_ref[0] = nid + b_id * _N


def _knn(pcT):
    return pl.pallas_call(
        _knn_body,
        grid=(_B, _N // _QT),
        in_specs=[
            pl.BlockSpec((1, _DIMS, _QT), lambda b, i: (b, 0, i)),
            pl.BlockSpec((1, _DIMS, _N), lambda b, i: (b, 0, 0)),
        ],
        out_specs=pl.BlockSpec((1, _QT, _K), lambda b, i: (b, i, 0)),
        out_shape=jax.ShapeDtypeStruct((_B, _N, _K), jnp.int32),
    )(pcT, pcT)


@functools.cache
def _sc_gather():
    @functools.partial(
        pl.kernel,
        out_type=[
            jax.ShapeDtypeStruct((_M, 2 * _F), jnp.float32),
            jax.ShapeDtypeStruct((_M,), jnp.float32),
            jax.ShapeDtypeStruct((_M,), jnp.float32),
        ],
        mesh=plsc.VectorSubcoreMesh(
            core_axis_name="c", subcore_axis_name="s",
            num_cores=_NC, num_subcores=_NS,
        ),
        scratch_types=[
            pltpu.VMEM((_CH,), jnp.int32),
            pltpu.VMEM((_CH, _F), jnp.float32),
            pltpu.VMEM((2 * _B * _N,), jnp.float32),
            pltpu.VMEM((_CH,), jnp.float32),
            pltpu.VMEM((_CH,), jnp.float32),
            pltpu.SemaphoreType.DMA,
        ],
        compiler_params=pltpu.CompilerParams(needs_layout_passes=False),
    )
    def gather(tab_hbm, pc_hbm, idx_hbm, nf_hbm, npx_hbm, npy_hbm,
               idx_v, rows_v, pct_v, nx_v, ny_v, sem):
        wid = lax.axis_index("s") * _NC + lax.axis_index("c")
        pltpu.sync_copy(pc_hbm, pct_v)

        def chunk(c, _):
            row = wid * _CPW + c
            pltpu.sync_copy(idx_hbm.at[row], idx_v)
            cp1 = pltpu.async_copy(tab_hbm.at[idx_v], rows_v, sem)
            for j in range(_CH // 16):
                i16 = idx_v[pl.ds(j * 16, 16)] * 2
                nx_v[pl.ds(j * 16, 16)] = plsc.load_gather(pct_v, [i16])
                ny_v[pl.ds(j * 16, 16)] = plsc.load_gather(pct_v, [i16 + 1])
            cp1.wait()
            pltpu.sync_copy(rows_v,
                            nf_hbm.at[pl.ds(row * _CH, _CH), pl.ds(0, _F)])
            pltpu.sync_copy(nx_v, npx_hbm.at[pl.ds(row * _CH, _CH)])
            pltpu.sync_copy(ny_v, npy_hbm.at[pl.ds(row * _CH, _CH)])
            return ()

        lax.fori_loop(0, _CPW, chunk, (), unroll=False)

    return gather


def _mlp_body(pcq_ref, npx_ref, npy_ref, w_ref, b_ref, _nf_ref, r_ref):
    qx = pcq_ref[0, 0, :]
    qy = pcq_ref[0, 1, :]
    nx = npx_ref[...]
    ny = npy_ref[...]
    qxb = jnp.broadcast_to(qx[:, None], (_QT, _K))
    qyb = jnp.broadcast_to(qy[:, None], (_QT, _K))
    relx = qxb - nx
    rely = qyb - ny
    norm = jnp.sqrt(relx * relx + rely * rely + jnp.float32(1e-12))

    feats7 = (qxb, qyb, nx, ny, relx, rely, norm)
    acc = jnp.broadcast_to(b_ref[0][None, None, :], (_QT, _K, _F))
    for c in range(7):
        acc = acc + feats7[c][:, :, None] * w_ref[c][None, None, :]
    r_ref[0] = jnp.maximum(acc, jnp.float32(0.0))


def _mlp(pcT, npx, npy, W, b2, nf_full):
    qspec = pl.BlockSpec((_QT, _K), lambda b, i: (b * (_N // _QT) + i, 0))
    return pl.pallas_call(
        _mlp_body,
        grid=(_B, _N // _QT),
        in_specs=[
            pl.BlockSpec((1, _DIMS, _QT), lambda b, i: (b, 0, i)),
            qspec,
            qspec,
            pl.BlockSpec((7, _F), lambda b, i: (0, 0)),
            pl.BlockSpec((1, _F), lambda b, i: (0, 0)),
            pl.BlockSpec(memory_space=pl.ANY),
        ],
        out_specs=pl.BlockSpec((1, _QT, _K, _F), lambda b, i: (b, i, 0, 1)),
        out_shape=jax.ShapeDtypeStruct((_B, _N, _K, 2 * _F), jnp.float32),
        input_output_aliases={5: 0},
    )(pcT, npx, npy, W, b2, nf_full)


def kernel(pc, feats, W, b):
    pcT = pc.transpose(0, 2, 1)
    nid = _knn(pcT)
    tab = feats.reshape(_B * _N, _F)
    pc_flat = pc.reshape(2 * _B * _N)
    idx2 = nid.reshape(_M // _CH, _CH)
    nf_full, npx, npy = _sc_gather()(tab, pc_flat, idx2)
    return _mlp(pcT, npx.reshape(_B * _N, _K), npy.reshape(_B * _N, _K),
                W, b.reshape(1, _F),
                nf_full.reshape(_B, _N, _K, 2 * _F))
